# baseline (speedup 1.0000x reference)
import functools

import jax
import jax.numpy as jnp
from jax import lax
from jax.experimental import pallas as pl
from jax.experimental.pallas import tpu as pltpu
from jax.experimental.pallas import tpu_sc as plsc

B, L, D = 4096, 200, 128
VOCAB_ROWS = 100000
H = D // 2
NC, NS = 2, 16
NW = NC * NS
BPW = B // NW
LANE = 16
NCH = D // LANE
NQ = H // LANE
G0 = 128
G1 = L - G0

_mesh = plsc.VectorSubcoreMesh(
    core_axis_name="c", subcore_axis_name="s", num_cores=NC, num_subcores=NS
)


def _make_embed_sum(nrows, row0, packed_mode):
    bpw = nrows // NW
    assert nrows % (NW * 4) == 0
    wpr = H if packed_mode else D

    @functools.partial(
        pl.kernel,
        out_type=jax.ShapeDtypeStruct((nrows, D), jnp.float32),
        mesh=_mesh,
        compiler_params=pltpu.CompilerParams(
            needs_layout_passes=False, use_tc_tiling_on_sc=False
        ),
        scratch_types=[
            pltpu.VMEM((bpw, L), jnp.int32),
            pltpu.VMEM(
                (4, L, wpr), jnp.int32 if packed_mode else jnp.float32
            ),
            pltpu.VMEM((bpw, D), jnp.float32),
            pltpu.SemaphoreType.DMA,
            pltpu.SemaphoreType.DMA,
            pltpu.SemaphoreType.DMA,
            pltpu.SemaphoreType.DMA,
        ],
    )
    def embed_sum(
        body_hbm, table_hbm, out_hbm, idx_v, rows_v, acc_v, sem0, sem1, sem2, sem3
    ):
        wid = lax.axis_index("s") * NC + lax.axis_index("c")
        base = wid * bpw
        sems = (sem0, sem1, sem2, sem3)

        pltpu.sync_copy(body_hbm.at[pl.ds(row0 + base, bpw)], idx_v)

        def start(i, bi):
            pltpu.async_copy(
                table_hbm.at[idx_v.at[i, pl.ds(0, G0)]],
                rows_v.at[bi, pl.ds(0, G0)],
                sems[bi],
            )
            pltpu.async_copy(
                table_hbm.at[idx_v.at[i, pl.ds(G0, G1)]],
                rows_v.at[bi, pl.ds(G0, G1)],
                sems[bi],
            )

        def wait(bi):
            pltpu.make_async_copy(
                table_hbm.at[idx_v.at[0, pl.ds(0, G0)]],
                rows_v.at[bi, pl.ds(0, G0)],
                sems[bi],
            ).wait()
            pltpu.make_async_copy(
                table_hbm.at[idx_v.at[0, pl.ds(G0, G1)]],
                rows_v.at[bi, pl.ds(G0, G1)],
                sems[bi],
            ).wait()

        start(0, 0)
        start(1, 1)
        start(2, 2)
        mask_hi = jnp.int32(-65536)

        @pl.loop(0, bpw, step=4)
        def _outer(i0):
            for b in range(4):
                i = i0 + b

                @pl.when(i + 3 < bpw)
                def _():
                    start(i + 3, (b + 3) % 4)

                wait(b)

                if packed_mode:

                    def red(l, acc):
                        new = list(acc)
                        for q in range(NQ):
                            pair = rows_v[b, l, pl.ds(LANE * q, LANE)]
                            hi = plsc.bitcast(pair & mask_hi, jnp.float32)
                            lo = plsc.bitcast(pair << 16, jnp.float32)
                            new[q] = acc[q] + hi
                            new[NQ + q] = acc[NQ + q] + lo
                        return tuple(new)

                else:

                    def red(l, acc):
                        return tuple(
                            acc[d] + rows_v[b, l, pl.ds(LANE * d, LANE)]
                            for d in range(NCH)
                        )

                acc = lax.fori_loop(
                    0, L, red,
                    tuple(jnp.zeros((LANE,), jnp.float32) for _ in range(NCH)),
                    unroll=4,
                )
                if packed_mode:
                    for q in range(NQ):
                        acc_v[i, pl.ds(LANE * q, LANE)] = acc[q]
                        acc_v[i, pl.ds(H + LANE * q, LANE)] = acc[NQ + q]
                else:
                    for d in range(NCH):
                        acc_v[i, pl.ds(LANE * d, LANE)] = acc[d]

        pltpu.sync_copy(acc_v, out_hbm.at[pl.ds(base, bpw)])

    return embed_sum


M_F32 = 768
_embed_sum_f32 = _make_embed_sum(M_F32, 0, packed_mode=False)
_embed_sum_bf16 = _make_embed_sum(B - M_F32, M_F32, packed_mode=True)


def _finish_body(
    sums_a_ref, sums_b_ref, body_ref, emb0f_ref, emb0b_ref, gamma_ref,
    beta_ref, out_ref
):
    body = body_ref[...]
    npos = jnp.sum((body > 0).astype(jnp.float32), axis=1, keepdims=True)
    nzero = jnp.float32(L) - npos
    correction = jnp.concatenate(
        [
            nzero[:M_F32] * emb0f_ref[...],
            nzero[M_F32:] * emb0b_ref[...],
        ],
        axis=0,
    )
    sums = jnp.concatenate([sums_a_ref[...], sums_b_ref[...]], axis=0)
    pooled = (sums - correction) / jnp.maximum(npos, 1.0)
    mu = jnp.mean(pooled, axis=0, keepdims=True)
    cen = pooled - mu
    var = jnp.mean(cen * cen, axis=0, keepdims=True)
    out_ref[...] = gamma_ref[...] * cen * lax.rsqrt(var + 1e-5) + beta_ref[...]


_PACK_BLK = 10000
_VHALF = VOCAB_ROWS // 2


def _pack_words(x):
    u = lax.bitcast_convert_type(x, jnp.uint32)
    r = u + jnp.uint32(0x7FFF) + ((u >> 16) & jnp.uint32(1))
    hi = r[:, :H] & jnp.uint32(0xFFFF0000)
    lo = r[:, H:] >> 16
    return lax.bitcast_convert_type(hi | lo, jnp.int32)


def _pack_body(xa_ref, xb_ref, out_ref):
    out_ref[...] = jnp.concatenate(
        [_pack_words(xa_ref[...]), _pack_words(xb_ref[...])], axis=1
    )


def _pack_table(emb_table):
    packed2 = pl.pallas_call(
        _pack_body,
        grid=(_VHALF // _PACK_BLK,),
        in_specs=[
            pl.BlockSpec((_PACK_BLK, D), lambda i: (i, 0)),
            pl.BlockSpec((_PACK_BLK, D), lambda i: (i + _VHALF // _PACK_BLK, 0)),
        ],
        out_specs=pl.BlockSpec((_PACK_BLK, D), lambda i: (i, 0)),
        out_shape=jax.ShapeDtypeStruct((_VHALF, D), jnp.int32),
    )(emb_table, emb_table)
    return packed2.reshape(VOCAB_ROWS, H)


def _unpack_row(packed_row):
    hi = lax.bitcast_convert_type(
        packed_row & jnp.int32(-65536), jnp.float32
    )
    lo = lax.bitcast_convert_type(packed_row << 16, jnp.float32)
    return jnp.concatenate([hi, lo], axis=1)


def kernel(title, body, emb_table, gamma, beta):
    del title
    body = body.astype(jnp.int32)
    sums_a = _embed_sum_f32(body, emb_table)
    packed = _pack_table(emb_table)
    slots = jnp.where(body < _VHALF, 2 * body, 2 * body - (VOCAB_ROWS - 1))
    sums_b = _embed_sum_bf16(slots, packed)
    emb0f = emb_table[0:1]
    emb0b = _unpack_row(packed[0:1])
    out = pl.pallas_call(
        _finish_body,
        out_shape=jax.ShapeDtypeStruct((B, D), jnp.float32),
    )(sums_a, sums_b, body, emb0f, emb0b, gamma.reshape(1, D), beta.reshape(1, D))
    return out


# speedup vs baseline: 1.0241x; 1.0241x over previous
import functools

import jax
import jax.numpy as jnp
from jax import lax
from jax.experimental import pallas as pl
from jax.experimental.pallas import tpu as pltpu
from jax.experimental.pallas import tpu_sc as plsc

B, L, D = 4096, 200, 128
VOCAB_ROWS = 100000
H = D // 2
NC, NS = 2, 16
NW = NC * NS
BPW = B // NW
LANE = 16
NCH = D // LANE
NQ = H // LANE
G0 = 128
G1 = L - G0

_mesh = plsc.VectorSubcoreMesh(
    core_axis_name="c", subcore_axis_name="s", num_cores=NC, num_subcores=NS
)


def _make_embed_sum(nrows, row0, packed_mode):
    bpw = nrows // NW
    assert nrows % (NW * 4) == 0
    wpr = H if packed_mode else D

    @functools.partial(
        pl.kernel,
        out_type=jax.ShapeDtypeStruct((nrows, D), jnp.float32),
        mesh=_mesh,
        compiler_params=pltpu.CompilerParams(
            needs_layout_passes=False, use_tc_tiling_on_sc=False
        ),
        scratch_types=[
            pltpu.VMEM((bpw, L), jnp.int32),
            pltpu.VMEM(
                (4, L, wpr), jnp.int32 if packed_mode else jnp.float32
            ),
            pltpu.VMEM((bpw, D), jnp.float32),
            pltpu.SemaphoreType.DMA,
            pltpu.SemaphoreType.DMA,
            pltpu.SemaphoreType.DMA,
            pltpu.SemaphoreType.DMA,
        ],
    )
    def embed_sum(
        body_hbm, table_hbm, out_hbm, idx_v, rows_v, acc_v, sem0, sem1, sem2, sem3
    ):
        wid = lax.axis_index("s") * NC + lax.axis_index("c")
        base = wid * bpw
        sems = (sem0, sem1, sem2, sem3)

        pltpu.sync_copy(body_hbm.at[pl.ds(row0 + base, bpw)], idx_v)

        def start(i, bi):
            pltpu.async_copy(
                table_hbm.at[idx_v.at[i, pl.ds(0, G0)]],
                rows_v.at[bi, pl.ds(0, G0)],
                sems[bi],
            )
            pltpu.async_copy(
                table_hbm.at[idx_v.at[i, pl.ds(G0, G1)]],
                rows_v.at[bi, pl.ds(G0, G1)],
                sems[bi],
            )

        def wait(bi):
            pltpu.make_async_copy(
                table_hbm.at[idx_v.at[0, pl.ds(0, G0)]],
                rows_v.at[bi, pl.ds(0, G0)],
                sems[bi],
            ).wait()
            pltpu.make_async_copy(
                table_hbm.at[idx_v.at[0, pl.ds(G0, G1)]],
                rows_v.at[bi, pl.ds(G0, G1)],
                sems[bi],
            ).wait()

        start(0, 0)
        start(1, 1)
        start(2, 2)
        mask_hi = jnp.int32(-65536)

        @pl.loop(0, bpw, step=4)
        def _outer(i0):
            for b in range(4):
                i = i0 + b

                @pl.when(i + 3 < bpw)
                def _():
                    start(i + 3, (b + 3) % 4)

                wait(b)

                if packed_mode:

                    def red(l, acc):
                        new = list(acc)
                        for q in range(NQ):
                            pair = rows_v[b, l, pl.ds(LANE * q, LANE)]
                            hi = plsc.bitcast(pair & mask_hi, jnp.float32)
                            lo = plsc.bitcast(pair << 16, jnp.float32)
                            new[q] = acc[q] + hi
                            new[NQ + q] = acc[NQ + q] + lo
                        return tuple(new)

                else:

                    def red(l, acc):
                        return tuple(
                            acc[d] + rows_v[b, l, pl.ds(LANE * d, LANE)]
                            for d in range(NCH)
                        )

                acc = lax.fori_loop(
                    0, L, red,
                    tuple(jnp.zeros((LANE,), jnp.float32) for _ in range(NCH)),
                    unroll=4,
                )
                if packed_mode:
                    for q in range(NQ):
                        acc_v[i, pl.ds(LANE * q, LANE)] = acc[q]
                        acc_v[i, pl.ds(H + LANE * q, LANE)] = acc[NQ + q]
                else:
                    for d in range(NCH):
                        acc_v[i, pl.ds(LANE * d, LANE)] = acc[d]

        pltpu.sync_copy(acc_v, out_hbm.at[pl.ds(base, bpw)])

    return embed_sum


M_F32 = 512
_embed_sum_f32 = _make_embed_sum(M_F32, 0, packed_mode=False)
_embed_sum_bf16 = _make_embed_sum(B - M_F32, M_F32, packed_mode=True)


def _finish_body(
    sums_a_ref, sums_b_ref, body_ref, emb0f_ref, emb0b_ref, gamma_ref,
    beta_ref, out_ref
):
    body = body_ref[...]
    npos = jnp.sum((body > 0).astype(jnp.float32), axis=1, keepdims=True)
    nzero = jnp.float32(L) - npos
    correction = jnp.concatenate(
        [
            nzero[:M_F32] * emb0f_ref[...],
            nzero[M_F32:] * emb0b_ref[...],
        ],
        axis=0,
    )
    sums = jnp.concatenate([sums_a_ref[...], sums_b_ref[...]], axis=0)
    pooled = (sums - correction) / jnp.maximum(npos, 1.0)
    mu = jnp.mean(pooled, axis=0, keepdims=True)
    cen = pooled - mu
    var = jnp.mean(cen * cen, axis=0, keepdims=True)
    out_ref[...] = gamma_ref[...] * cen * lax.rsqrt(var + 1e-5) + beta_ref[...]


_PACK_BLK = 10000
_VHALF = VOCAB_ROWS // 2


def _pack_words(x):
    u = lax.bitcast_convert_type(x, jnp.uint32)
    r = u + jnp.uint32(0x7FFF) + ((u >> 16) & jnp.uint32(1))
    hi = r[:, :H] & jnp.uint32(0xFFFF0000)
    lo = r[:, H:] >> 16
    return lax.bitcast_convert_type(hi | lo, jnp.int32)


def _pack_body(xa_ref, xb_ref, out_ref):
    out_ref[...] = jnp.concatenate(
        [_pack_words(xa_ref[...]), _pack_words(xb_ref[...])], axis=1
    )


def _pack_table(emb_table):
    packed2 = pl.pallas_call(
        _pack_body,
        grid=(_VHALF // _PACK_BLK,),
        in_specs=[
            pl.BlockSpec((_PACK_BLK, D), lambda i: (i, 0)),
            pl.BlockSpec((_PACK_BLK, D), lambda i: (i + _VHALF // _PACK_BLK, 0)),
        ],
        out_specs=pl.BlockSpec((_PACK_BLK, D), lambda i: (i, 0)),
        out_shape=jax.ShapeDtypeStruct((_VHALF, D), jnp.int32),
    )(emb_table, emb_table)
    return packed2.reshape(VOCAB_ROWS, H)


def _unpack_row(packed_row):
    hi = lax.bitcast_convert_type(
        packed_row & jnp.int32(-65536), jnp.float32
    )
    lo = lax.bitcast_convert_type(packed_row << 16, jnp.float32)
    return jnp.concatenate([hi, lo], axis=1)


def kernel(title, body, emb_table, gamma, beta):
    del title
    body = body.astype(jnp.int32)
    sums_a = _embed_sum_f32(body, emb_table)
    packed = _pack_table(emb_table)
    slots = jnp.where(body < _VHALF, 2 * body, 2 * body - (VOCAB_ROWS - 1))
    sums_b = _embed_sum_bf16(slots, packed)
    emb0f = emb_table[0:1]
    emb0b = _unpack_row(packed[0:1])
    out = pl.pallas_call(
        _finish_body,
        out_shape=jax.ShapeDtypeStruct((B, D), jnp.float32),
    )(sums_a, sums_b, body, emb0f, emb0b, gamma.reshape(1, D), beta.reshape(1, D))
    return out


# speedup vs baseline: 1.0664x; 1.0413x over previous
import functools

import jax
import jax.numpy as jnp
from jax import lax
from jax.experimental import pallas as pl
from jax.experimental.pallas import tpu as pltpu
from jax.experimental.pallas import tpu_sc as plsc

B, L, D = 4096, 200, 128
VOCAB_ROWS = 100000
H = D // 2
NC, NS = 2, 16
NW = NC * NS
BPW = B // NW
LANE = 16
NCH = D // LANE
NQ = H // LANE
G0 = 128
G1 = L - G0

_mesh = plsc.VectorSubcoreMesh(
    core_axis_name="c", subcore_axis_name="s", num_cores=NC, num_subcores=NS
)


def _make_embed_sum(nrows, row0, packed_mode):
    bpw = nrows // NW
    assert nrows % (NW * 4) == 0
    wpr = H if packed_mode else D

    @functools.partial(
        pl.kernel,
        out_type=jax.ShapeDtypeStruct((nrows, D), jnp.float32),
        mesh=_mesh,
        compiler_params=pltpu.CompilerParams(
            needs_layout_passes=False, use_tc_tiling_on_sc=False
        ),
        scratch_types=[
            pltpu.VMEM((bpw, L), jnp.int32),
            pltpu.VMEM(
                (4, L, wpr), jnp.int32 if packed_mode else jnp.float32
            ),
            pltpu.VMEM((bpw, D), jnp.float32),
            pltpu.SemaphoreType.DMA,
            pltpu.SemaphoreType.DMA,
            pltpu.SemaphoreType.DMA,
            pltpu.SemaphoreType.DMA,
        ],
    )
    def embed_sum(
        body_hbm, table_hbm, out_hbm, idx_v, rows_v, acc_v, sem0, sem1, sem2, sem3
    ):
        wid = lax.axis_index("s") * NC + lax.axis_index("c")
        base = wid * bpw
        sems = (sem0, sem1, sem2, sem3)

        pltpu.sync_copy(body_hbm.at[pl.ds(row0 + base, bpw)], idx_v)

        def start(i, bi):
            pltpu.async_copy(
                table_hbm.at[idx_v.at[i, pl.ds(0, G0)]],
                rows_v.at[bi, pl.ds(0, G0)],
                sems[bi],
            )
            pltpu.async_copy(
                table_hbm.at[idx_v.at[i, pl.ds(G0, G1)]],
                rows_v.at[bi, pl.ds(G0, G1)],
                sems[bi],
            )

        def wait(bi):
            pltpu.make_async_copy(
                table_hbm.at[idx_v.at[0, pl.ds(0, G0)]],
                rows_v.at[bi, pl.ds(0, G0)],
                sems[bi],
            ).wait()
            pltpu.make_async_copy(
                table_hbm.at[idx_v.at[0, pl.ds(G0, G1)]],
                rows_v.at[bi, pl.ds(G0, G1)],
                sems[bi],
            ).wait()

        start(0, 0)
        start(1, 1)
        start(2, 2)
        mask_hi = jnp.int32(-65536)

        @pl.loop(0, bpw, step=4)
        def _outer(i0):
            for b in range(4):
                i = i0 + b

                @pl.when(i + 3 < bpw)
                def _():
                    start(i + 3, (b + 3) % 4)

                wait(b)

                if packed_mode:

                    def red(l, acc):
                        new = list(acc)
                        for q in range(NQ):
                            pair = rows_v[b, l, pl.ds(LANE * q, LANE)]
                            hi = plsc.bitcast(pair & mask_hi, jnp.float32)
                            lo = plsc.bitcast(pair << 16, jnp.float32)
                            new[q] = acc[q] + hi
                            new[NQ + q] = acc[NQ + q] + lo
                        return tuple(new)

                else:

                    def red(l, acc):
                        return tuple(
                            acc[d] + rows_v[b, l, pl.ds(LANE * d, LANE)]
                            for d in range(NCH)
                        )

                acc = lax.fori_loop(
                    0, L, red,
                    tuple(jnp.zeros((LANE,), jnp.float32) for _ in range(NCH)),
                    unroll=4,
                )
                if packed_mode:
                    for q in range(NQ):
                        acc_v[i, pl.ds(LANE * q, LANE)] = acc[q]
                        acc_v[i, pl.ds(H + LANE * q, LANE)] = acc[NQ + q]
                else:
                    for d in range(NCH):
                        acc_v[i, pl.ds(LANE * d, LANE)] = acc[d]

        pltpu.sync_copy(acc_v, out_hbm.at[pl.ds(base, bpw)])

    return embed_sum


_embed_sum_bf16 = _make_embed_sum(B, 0, packed_mode=True)


def _finish_body(sums_ref, body_ref, emb0_ref, gamma_ref, beta_ref, out_ref):
    body = body_ref[...]
    npos = jnp.sum((body > 0).astype(jnp.float32), axis=1, keepdims=True)
    nzero = jnp.float32(L) - npos
    pooled = (sums_ref[...] - nzero * emb0_ref[...]) / jnp.maximum(npos, 1.0)
    mu = jnp.mean(pooled, axis=0, keepdims=True)
    cen = pooled - mu
    var = jnp.mean(cen * cen, axis=0, keepdims=True)
    out_ref[...] = gamma_ref[...] * cen * lax.rsqrt(var + 1e-5) + beta_ref[...]


_PACK_BLK = 10000
_VHALF = VOCAB_ROWS // 2


def _pack_words(x):
    u = lax.bitcast_convert_type(x, jnp.uint32)
    r = u + jnp.uint32(0x7FFF) + ((u >> 16) & jnp.uint32(1))
    hi = r[:, :H] & jnp.uint32(0xFFFF0000)
    lo = r[:, H:] >> 16
    return lax.bitcast_convert_type(hi | lo, jnp.int32)


def _pack_body(xa_ref, xb_ref, out_ref):
    out_ref[...] = jnp.concatenate(
        [_pack_words(xa_ref[...]), _pack_words(xb_ref[...])], axis=1
    )


def _pack_table(emb_table):
    packed2 = pl.pallas_call(
        _pack_body,
        grid=(_VHALF // _PACK_BLK,),
        in_specs=[
            pl.BlockSpec((_PACK_BLK, D), lambda i: (i, 0)),
            pl.BlockSpec((_PACK_BLK, D), lambda i: (i + _VHALF // _PACK_BLK, 0)),
        ],
        out_specs=pl.BlockSpec((_PACK_BLK, D), lambda i: (i, 0)),
        out_shape=jax.ShapeDtypeStruct((_VHALF, D), jnp.int32),
    )(emb_table, emb_table)
    return packed2.reshape(VOCAB_ROWS, H)


def _unpack_row(packed_row):
    hi = lax.bitcast_convert_type(
        packed_row & jnp.int32(-65536), jnp.float32
    )
    lo = lax.bitcast_convert_type(packed_row << 16, jnp.float32)
    return jnp.concatenate([hi, lo], axis=1)


def kernel(title, body, emb_table, gamma, beta):
    del title
    body = body.astype(jnp.int32)
    packed = _pack_table(emb_table)
    slots = jnp.where(body < _VHALF, 2 * body, 2 * body - (VOCAB_ROWS - 1))
    sums = _embed_sum_bf16(slots, packed)
    emb0 = _unpack_row(packed[0:1])
    out = pl.pallas_call(
        _finish_body,
        out_shape=jax.ShapeDtypeStruct((B, D), jnp.float32),
    )(sums, body, emb0, gamma.reshape(1, D), beta.reshape(1, D))
    return out


# speedup vs baseline: 1.0700x; 1.0034x over previous
import functools

import jax
import jax.numpy as jnp
from jax import lax
from jax.experimental import pallas as pl
from jax.experimental.pallas import tpu as pltpu
from jax.experimental.pallas import tpu_sc as plsc

B, L, D = 4096, 200, 128
VOCAB_ROWS = 100000
H = D // 2
NC, NS = 2, 16
NW = NC * NS
BPW = B // NW
LANE = 16
NCH = D // LANE
NQ = H // LANE
G0 = 128
G1 = L - G0

_mesh = plsc.VectorSubcoreMesh(
    core_axis_name="c", subcore_axis_name="s", num_cores=NC, num_subcores=NS
)


def _make_embed_sum(nrows, row0, packed_mode):
    bpw = nrows // NW
    assert nrows % (NW * 4) == 0
    wpr = H if packed_mode else D

    @functools.partial(
        pl.kernel,
        out_type=jax.ShapeDtypeStruct((nrows * D,), jnp.float32),
        mesh=_mesh,
        compiler_params=pltpu.CompilerParams(
            needs_layout_passes=False, use_tc_tiling_on_sc=False
        ),
        scratch_types=[
            pltpu.VMEM((bpw, L), jnp.int32),
            pltpu.VMEM(
                (4, L, wpr), jnp.int32 if packed_mode else jnp.float32
            ),
            pltpu.VMEM((bpw * D,), jnp.float32),
            pltpu.SemaphoreType.DMA,
            pltpu.SemaphoreType.DMA,
            pltpu.SemaphoreType.DMA,
            pltpu.SemaphoreType.DMA,
        ],
    )
    def embed_sum(
        body_hbm, table_hbm, out_hbm, idx_v, rows_v, acc_v, sem0, sem1, sem2, sem3
    ):
        wid = lax.axis_index("s") * NC + lax.axis_index("c")
        base = wid * bpw
        sems = (sem0, sem1, sem2, sem3)

        pltpu.sync_copy(body_hbm.at[pl.ds(row0 + base, bpw)], idx_v)

        def start(i, bi):
            pltpu.async_copy(
                table_hbm.at[idx_v.at[i, pl.ds(0, G0)]],
                rows_v.at[bi, pl.ds(0, G0)],
                sems[bi],
            )
            pltpu.async_copy(
                table_hbm.at[idx_v.at[i, pl.ds(G0, G1)]],
                rows_v.at[bi, pl.ds(G0, G1)],
                sems[bi],
            )

        def wait(bi):
            pltpu.make_async_copy(
                table_hbm.at[idx_v.at[0, pl.ds(0, G0)]],
                rows_v.at[bi, pl.ds(0, G0)],
                sems[bi],
            ).wait()
            pltpu.make_async_copy(
                table_hbm.at[idx_v.at[0, pl.ds(G0, G1)]],
                rows_v.at[bi, pl.ds(G0, G1)],
                sems[bi],
            ).wait()

        start(0, 0)
        start(1, 1)
        start(2, 2)
        mask_hi = jnp.int32(-65536)

        @pl.loop(0, bpw, step=4)
        def _outer(i0):
            for b in range(4):
                i = i0 + b

                @pl.when(i + 3 < bpw)
                def _():
                    start(i + 3, (b + 3) % 4)

                wait(b)

                if packed_mode:

                    def red(l, acc):
                        new = list(acc)
                        for q in range(NQ):
                            pair = rows_v[b, l, pl.ds(LANE * q, LANE)]
                            hi = plsc.bitcast(pair & mask_hi, jnp.float32)
                            lo = plsc.bitcast(pair << 16, jnp.float32)
                            new[q] = acc[q] + hi
                            new[NQ + q] = acc[NQ + q] + lo
                        return tuple(new)

                else:

                    def red(l, acc):
                        return tuple(
                            acc[d] + rows_v[b, l, pl.ds(LANE * d, LANE)]
                            for d in range(NCH)
                        )

                acc = lax.fori_loop(
                    0, L, red,
                    tuple(jnp.zeros((LANE,), jnp.float32) for _ in range(NCH)),
                    unroll=4,
                )
                if packed_mode:
                    for q in range(NQ):
                        acc_v[pl.ds(i * D + LANE * q, LANE)] = acc[q]
                        acc_v[pl.ds(i * D + H + LANE * q, LANE)] = acc[NQ + q]
                else:
                    for d in range(NCH):
                        acc_v[pl.ds(i * D + LANE * d, LANE)] = acc[d]

        pltpu.sync_copy(acc_v, out_hbm.at[pl.ds(base * D, bpw * D)])

    return embed_sum


_embed_sum_bf16 = _make_embed_sum(B, 0, packed_mode=True)


def _finish_body(sums_ref, body_ref, emb0_ref, gamma_ref, beta_ref, out_ref):
    body = body_ref[...]
    npos = jnp.sum((body > 0).astype(jnp.float32), axis=1, keepdims=True)
    nzero = jnp.float32(L) - npos
    pooled = (sums_ref[...] - nzero * emb0_ref[...]) / jnp.maximum(npos, 1.0)
    mu = jnp.mean(pooled, axis=0, keepdims=True)
    cen = pooled - mu
    var = jnp.mean(cen * cen, axis=0, keepdims=True)
    out_ref[...] = gamma_ref[...] * cen * lax.rsqrt(var + 1e-5) + beta_ref[...]


_PACK_BLK = 10000
_VHALF = VOCAB_ROWS // 2


def _pack_words(x):
    u = lax.bitcast_convert_type(x, jnp.uint32)
    r = u + jnp.uint32(0x7FFF) + ((u >> 16) & jnp.uint32(1))
    hi = r[:, :H] & jnp.uint32(0xFFFF0000)
    lo = r[:, H:] >> 16
    return lax.bitcast_convert_type(hi | lo, jnp.int32)


def _pack_body(xa_ref, xb_ref, out_ref):
    out_ref[...] = jnp.concatenate(
        [_pack_words(xa_ref[...]), _pack_words(xb_ref[...])], axis=1
    )


def _pack_table(emb_table):
    packed2 = pl.pallas_call(
        _pack_body,
        grid=(_VHALF // _PACK_BLK,),
        in_specs=[
            pl.BlockSpec((_PACK_BLK, D), lambda i: (i, 0)),
            pl.BlockSpec((_PACK_BLK, D), lambda i: (i + _VHALF // _PACK_BLK, 0)),
        ],
        out_specs=pl.BlockSpec((_PACK_BLK, D), lambda i: (i, 0)),
        out_shape=jax.ShapeDtypeStruct((_VHALF, D), jnp.int32),
    )(emb_table, emb_table)
    return packed2.reshape(VOCAB_ROWS, H)


def _unpack_row(packed_row):
    hi = lax.bitcast_convert_type(
        packed_row & jnp.int32(-65536), jnp.float32
    )
    lo = lax.bitcast_convert_type(packed_row << 16, jnp.float32)
    return jnp.concatenate([hi, lo], axis=1)


def kernel(title, body, emb_table, gamma, beta):
    del title
    body = body.astype(jnp.int32)
    packed = _pack_table(emb_table)
    slots = jnp.where(body < _VHALF, 2 * body, 2 * body - (VOCAB_ROWS - 1))
    sums = _embed_sum_bf16(slots, packed).reshape(B, D)
    emb0 = _unpack_row(packed[0:1])
    out = pl.pallas_call(
        _finish_body,
        out_shape=jax.ShapeDtypeStruct((B, D), jnp.float32),
    )(sums, body, emb0, gamma.reshape(1, D), beta.reshape(1, D))
    return out
